# baseline (speedup 1.0000x reference)
import jax
import jax.numpy as jnp
from jax.experimental import pallas as pl
from jax.experimental.pallas import tpu as pltpu

_LAMBDA = 0.7
_CHUNKS = 4


def _elu(v):
    return jnp.where(v > 0, v, jnp.exp(v) - 1.0)


def _col_softmax(s):
    e = jnp.exp(s - jnp.max(s))
    return e / jnp.sum(e)


def _dot_t(a, b):
    return jax.lax.dot_general(a, b, (((1,), (1,)), ((), ())))


def _fused_body(adj_hbm, x_ref, fc1_ref, aw1_ref, fc2_ref, aw2_ref,
                sgcw_ref, b_ref, out_ref,
                rhs_ref, t1_ref, v_ref, hp_ref, ns_ref):
    adj_ref = adj_hbm
    n = adj_ref.shape[0]
    rb = n // _CHUNKS
    x = x_ref[...]
    f = x.shape[1]
    hidden = fc1_ref.shape[0]
    out_f = b_ref.shape[1]

    h = _dot_t(x, fc1_ref[...])
    g1 = _col_softmax(_dot_t(h, aw1_ref[:, hidden:])) * h
    u1 = _dot_t(x, sgcw_ref[:, :f])
    u3 = _dot_t(x, sgcw_ref[:, 2 * f:])
    rhs_ref[...] = jnp.concatenate([u1, u3, g1], axis=1)

    c1 = 2 * out_f
    for c in range(_CHUNKS):
        rows = pl.ds(c * rb, rb)
        m1 = jnp.dot(adj_ref[rows, :], rhs_ref[...])
        t1_ref[rows, :] = m1[:, :out_f]
        v_ref[rows, :out_f] = m1[:, out_f:c1]
        hp_ref[rows, :] = _elu(m1[:, c1:c1 + hidden])

    h2 = _dot_t(hp_ref[...], fc2_ref[...])
    g2 = _col_softmax(h2 * aw2_ref[0, 1]) * h2
    v_ref[:, out_f:] = g2

    for c in range(_CHUNKS):
        rows = pl.ds(c * rb, rb)
        m2 = jnp.dot(adj_ref[rows, :], v_ref[...])
        ns_ref[rows, :] = _elu(m2[:, out_f:])
        v_ref2 = m2[:, :out_f]
        rhs_ref[rows, :out_f] = v_ref2

    for c in range(_CHUNKS):
        rows = pl.ds(c * rb, rb)
        m3 = jnp.dot(adj_ref[rows, :], rhs_ref[:, :out_f])
        keep = (ns_ref[rows, :] > _LAMBDA).astype(jnp.float32)
        out_ref[rows, :] = (keep * t1_ref[rows, :] + (1.0 - keep) * m3
                            + b_ref[...])


def kernel(x, adj, gat1_fc_w, gat1_attn_w, gat2_fc_w, gat2_attn_w,
           sgc_w, sgc_b):
    n, f = x.shape
    hidden = gat1_fc_w.shape[0]
    out_f = sgc_w.shape[0]

    vmem = pl.BlockSpec(memory_space=pltpu.MemorySpace.VMEM)
    return pl.pallas_call(
        _fused_body,
        in_specs=[vmem] * 8,
        out_specs=vmem,
        out_shape=jax.ShapeDtypeStruct((n, out_f), jnp.float32),
        scratch_shapes=[
            pltpu.VMEM((n, 2 * out_f + hidden), jnp.float32),
            pltpu.VMEM((n, out_f), jnp.float32),
            pltpu.VMEM((n, out_f + 1), jnp.float32),
            pltpu.VMEM((n, hidden), jnp.float32),
            pltpu.VMEM((n, 1), jnp.float32),
        ],
        compiler_params=pltpu.CompilerParams(
            vmem_limit_bytes=60 * 1024 * 1024),
    )(adj, x, gat1_fc_w, gat1_attn_w, gat2_fc_w, gat2_attn_w, sgc_w,
      sgc_b.reshape(1, out_f))


# speedup vs baseline: 1.0526x; 1.0526x over previous
import jax
import jax.numpy as jnp
from jax.experimental import pallas as pl
from jax.experimental.pallas import tpu as pltpu

_LAMBDA = 0.7
_CHUNKS = 4


def _elu(v):
    return jnp.where(v > 0, v, jnp.exp(v) - 1.0)


def _col_softmax(s):
    e = jnp.exp(s - jnp.max(s))
    return e / jnp.sum(e)


def _dot_t(a, b):
    return jax.lax.dot_general(a, b, (((1,), (1,)), ((), ())))


def _fused_body(adj_hbm, x_ref, fc1_ref, aw1_ref, fc2_ref, aw2_ref,
                sgcw_ref, b_ref, out_ref,
                adj_ref, rhs_ref, t1_ref, v_ref, hp_ref, ns_ref, sems):
    n = adj_ref.shape[0]
    rb = n // _CHUNKS

    def _chunk_copy(c):
        rows = pl.ds(c * rb, rb)
        return pltpu.make_async_copy(adj_hbm.at[rows, :],
                                     adj_ref.at[rows, :], sems.at[c])

    for c in range(_CHUNKS):
        _chunk_copy(c).start()

    x = x_ref[...]
    f = x.shape[1]
    hidden = fc1_ref.shape[0]
    out_f = b_ref.shape[1]

    h = _dot_t(x, fc1_ref[...])
    g1 = _col_softmax(_dot_t(h, aw1_ref[:, hidden:])) * h
    u1 = _dot_t(x, sgcw_ref[:, :f])
    u3 = _dot_t(x, sgcw_ref[:, 2 * f:])
    rhs_ref[...] = jnp.concatenate([u1, u3, g1], axis=1)

    c1 = 2 * out_f
    for c in range(_CHUNKS):
        _chunk_copy(c).wait()
        rows = pl.ds(c * rb, rb)
        m1 = jnp.dot(adj_ref[rows, :], rhs_ref[...])
        t1_ref[rows, :] = m1[:, :out_f]
        v_ref[rows, :out_f] = m1[:, out_f:c1]
        hp_ref[rows, :] = _elu(m1[:, c1:c1 + hidden])

    h2 = _dot_t(hp_ref[...], fc2_ref[...])
    g2 = _col_softmax(h2 * aw2_ref[0, 1]) * h2
    v_ref[:, out_f:] = g2

    for c in range(_CHUNKS):
        rows = pl.ds(c * rb, rb)
        m2 = jnp.dot(adj_ref[rows, :], v_ref[...])
        ns_ref[rows, :] = _elu(m2[:, out_f:])
        v_ref2 = m2[:, :out_f]
        rhs_ref[rows, :out_f] = v_ref2

    for c in range(_CHUNKS):
        rows = pl.ds(c * rb, rb)
        m3 = jnp.dot(adj_ref[rows, :], rhs_ref[:, :out_f])
        keep = (ns_ref[rows, :] > _LAMBDA).astype(jnp.float32)
        out_ref[rows, :] = (keep * t1_ref[rows, :] + (1.0 - keep) * m3
                            + b_ref[...])


def kernel(x, adj, gat1_fc_w, gat1_attn_w, gat2_fc_w, gat2_attn_w,
           sgc_w, sgc_b):
    n, f = x.shape
    hidden = gat1_fc_w.shape[0]
    out_f = sgc_w.shape[0]

    vmem = pl.BlockSpec(memory_space=pltpu.MemorySpace.VMEM)
    return pl.pallas_call(
        _fused_body,
        in_specs=[pl.BlockSpec(memory_space=pltpu.MemorySpace.HBM)]
        + [vmem] * 7,
        out_specs=vmem,
        out_shape=jax.ShapeDtypeStruct((n, out_f), jnp.float32),
        scratch_shapes=[
            pltpu.VMEM((n, n), jnp.float32),
            pltpu.VMEM((n, 2 * out_f + hidden), jnp.float32),
            pltpu.VMEM((n, out_f), jnp.float32),
            pltpu.VMEM((n, out_f + 1), jnp.float32),
            pltpu.VMEM((n, hidden), jnp.float32),
            pltpu.VMEM((n, 1), jnp.float32),
            pltpu.SemaphoreType.DMA((_CHUNKS,)),
        ],
        compiler_params=pltpu.CompilerParams(
            vmem_limit_bytes=60 * 1024 * 1024),
    )(adj, x, gat1_fc_w, gat1_attn_w, gat2_fc_w, gat2_attn_w, sgc_w,
      sgc_b.reshape(1, out_f))
